# baseline (speedup 1.0000x reference)
import jax
import jax.numpy as jnp
from jax import lax
from jax.experimental import pallas as pl
from jax.experimental.pallas import tpu as pltpu
from jax.experimental.pallas import tpu_sc as plsc

NUM_CORES = 2
NUM_SUBCORES = 16
NW = NUM_CORES * NUM_SUBCORES

T = 26
HIDDEN = 128
CHUNK = 128
N_TOTAL = 100000
SC_CHUNKS_PER_WORKER = 15
SC_CHUNKS = NW * SC_CHUNKS_PER_WORKER
N_SC = SC_CHUNKS * CHUNK
N_TC = N_TOTAL - N_SC
TC_BLOCK = 128
SLOTS_PER_WORKER = SC_CHUNKS_PER_WORKER * CHUNK


def _combine_body(w0_ref, w1_ref, w2_ref, xt_ref, s01_ref, c01_ref, c2_ref):
    w0, w1 = w0_ref[...], w1_ref[...]
    s01_ref[...] = (w0[:, None, :] + w1[None, :, :]).reshape(T * T, HIDDEN)
    xt = xt_ref[...]
    c01_ref[...] = T * xt[0] + xt[1]
    c2_ref[...] = xt[2]
    del w2_ref


@jax.jit
def _combine(w0, w1, w2, xt3):
    return pl.pallas_call(
        _combine_body,
        out_shape=(
            jax.ShapeDtypeStruct((T * T, HIDDEN), jnp.float32),
            jax.ShapeDtypeStruct(xt3.shape[1:], jnp.int32),
            jax.ShapeDtypeStruct(xt3.shape[1:], jnp.int32),
        ),
    )(w0, w1, w2, xt3)


def _sc_body(c01_hbm, c2_hbm, w2_hbm, s01_hbm, out_hbm,
             idx01_v, idx2_v, buf0, buf1, buf2, s01_sp, w2_sp,
             sem_g, sem_a, sem_w0, sem_w1, sem_w2):
    core = lax.axis_index("c")
    sid = lax.axis_index("s")
    wid = sid * NUM_CORES + core
    sbase = pl.multiple_of(wid * SLOTS_PER_WORKER, CHUNK)

    @pl.when(sid == 0)
    def _():
        pltpu.sync_copy(s01_hbm, s01_sp)
        pltpu.sync_copy(w2_hbm, w2_sp)

    pltpu.sync_copy(c01_hbm.at[pl.ds(sbase, SLOTS_PER_WORKER)], idx01_v)
    pltpu.sync_copy(c2_hbm.at[pl.ds(sbase, SLOTS_PER_WORKER)], idx2_v)
    plsc.subcore_barrier()

    bufs = (buf0, buf1, buf2)
    wsems = (sem_w0, sem_w1, sem_w2)

    def gather01(c, s):
        pltpu.async_copy(
            s01_sp.at[idx01_v.at[pl.ds(c * CHUNK, CHUNK)]], bufs[s], sem_g)

    def wait_g01(s):
        pltpu.make_async_copy(
            out_hbm.at[pl.ds(0, CHUNK), :], bufs[s], sem_g).wait()

    def gather2_add(c, s):
        return pltpu.async_copy(
            w2_sp.at[idx2_v.at[pl.ds(c * CHUNK, CHUNK)]], bufs[s],
            sem_a, add=True)

    def write(c, s):
        pltpu.async_copy(
            bufs[s], out_hbm.at[pl.ds(sbase + c * CHUNK, CHUNK), :], wsems[s])

    def wait_write(s):
        pltpu.make_async_copy(
            bufs[s], out_hbm.at[pl.ds(0, CHUNK), :], wsems[s]).wait()

    def step(c, s, wait_prev_write, prefetch=True):
        wait_g01(s)
        h_a = gather2_add(c, s)
        nxt = (s + 1) % 3
        if wait_prev_write:
            wait_write(nxt)
        if prefetch:
            gather01(c + 1, nxt)
        h_a.wait()
        write(c, s)

    gather01(0, 0)
    step(0, 0, False)
    step(1, 1, False)
    step(2, 2, True)

    def body(k, carry):
        c = 3 * k
        step(c, 0, True)
        step(c + 1, 1, True)
        step(c + 2, 2, True)
        return carry

    lax.fori_loop(1, 4, body, 0)

    step(12, 0, True)
    step(13, 1, True)
    step(14, 2, True, prefetch=False)

    wait_write(1)
    wait_write(2)


@jax.jit
def _encode(c01, c2, w2, s01):
    mesh = plsc.VectorSubcoreMesh(core_axis_name="c", subcore_axis_name="s")
    return pl.kernel(
        _sc_body,
        out_type=jax.ShapeDtypeStruct((N_TOTAL, HIDDEN), jnp.float32),
        mesh=mesh,
        scratch_types=[
            pltpu.VMEM((SLOTS_PER_WORKER,), jnp.int32),
            pltpu.VMEM((SLOTS_PER_WORKER,), jnp.int32),
            pltpu.VMEM((CHUNK, HIDDEN), jnp.float32),
            pltpu.VMEM((CHUNK, HIDDEN), jnp.float32),
            pltpu.VMEM((CHUNK, HIDDEN), jnp.float32),
            pltpu.VMEM_SHARED((T * T, HIDDEN), jnp.float32),
            pltpu.VMEM_SHARED((T, HIDDEN), jnp.float32),
            pltpu.SemaphoreType.DMA,
            pltpu.SemaphoreType.DMA,
            pltpu.SemaphoreType.DMA,
            pltpu.SemaphoreType.DMA,
            pltpu.SemaphoreType.DMA,
        ],
    )(c01, c2, w2, s01)


def _tc_fill_body(out_in_ref, xt_ref, w0_ref, w1_ref, w2_ref, out_ref):
    acc = jnp.zeros((TC_BLOCK, HIDDEN), jnp.float32)
    for t, w_ref in ((0, w0_ref), (1, w1_ref), (2, w2_ref)):
        col = xt_ref[t]
        onehot = (col[:, None] == lax.broadcasted_iota(
            jnp.int32, (TC_BLOCK, T), 1)).astype(jnp.float32)
        acc = acc + jnp.dot(onehot, w_ref[...],
                            precision=lax.Precision.HIGHEST,
                            preferred_element_type=jnp.float32)
    out_ref[...] = acc
    del out_in_ref


@jax.jit
def _tc_fill(out_sc, xt_tc, w0, w1, w2):
    grid = (N_TC + TC_BLOCK - 1) // TC_BLOCK
    blk = pl.BlockSpec((TC_BLOCK, HIDDEN),
                       lambda j: (N_SC // TC_BLOCK + j, 0))
    wspec = pl.BlockSpec((T, HIDDEN), lambda j: (0, 0))
    return pl.pallas_call(
        _tc_fill_body,
        grid=(grid,),
        in_specs=[
            blk,
            pl.BlockSpec((3, TC_BLOCK), lambda j: (0, j)),
            wspec, wspec, wspec,
        ],
        out_specs=blk,
        out_shape=jax.ShapeDtypeStruct((N_TOTAL, HIDDEN), jnp.float32),
        input_output_aliases={0: 0},
    )(out_sc, xt_tc, w0, w1, w2)


def kernel(x, W0, W1, W2):
    if x.ndim == 1:
        x = x[:, None]
    xt = x.T.astype(jnp.int32)
    xt_sc = xt[:, :N_SC].reshape(3, N_SC // HIDDEN, HIDDEN)
    s01, c01, c2 = _combine(W0, W1, W2, xt_sc)
    out_sc = _encode(c01.reshape(N_SC), c2.reshape(N_SC), W2, s01)
    return _tc_fill(out_sc, xt[:, N_SC:], W0, W1, W2)


# speedup vs baseline: 3.7394x; 3.7394x over previous
import jax
import jax.numpy as jnp
from jax import lax
from jax.experimental import pallas as pl
from jax.experimental.pallas import tpu as pltpu
from jax.experimental.pallas import tpu_sc as plsc

NUM_CORES = 2
NUM_SUBCORES = 16
NW = NUM_CORES * NUM_SUBCORES

T = 26
HIDDEN = 128
CHUNK = 128
N_TOTAL = 100000
NUM_CHUNKS = (N_TOTAL + CHUNK - 1) // CHUNK
BIG_WORKERS = NUM_CHUNKS - 24 * NW
MAIN_CHUNKS = 24
SLOTS_PER_WORKER = 25 * CHUNK
NP = 102400
LAST_BASE = N_TOTAL - CHUNK


def _combine_body(w0_ref, w1_ref, w2_ref, xt_ref, s01_ref, c01_ref, c2_ref):
    w0, w1 = w0_ref[...], w1_ref[...]
    s01_ref[...] = (w0[:, None, :] + w1[None, :, :]).reshape(T * T, HIDDEN)
    xt = xt_ref[...]
    c01_ref[...] = T * xt[0] + xt[1]
    c2_ref[...] = xt[2]
    del w2_ref


@jax.jit
def _combine(w0, w1, w2, xt3):
    return pl.pallas_call(
        _combine_body,
        out_shape=(
            jax.ShapeDtypeStruct((T * T, HIDDEN), jnp.float32),
            jax.ShapeDtypeStruct(xt3.shape[1:], jnp.int32),
            jax.ShapeDtypeStruct(xt3.shape[1:], jnp.int32),
        ),
    )(w0, w1, w2, xt3)


def _sc_body(c01_hbm, c2_hbm, w2_hbm, s01_hbm, out_hbm,
             idx01_v, idx2_v, buf0, buf1, buf2, s01_sp, w2_sp,
             sem_g, sem_a, sem_w0, sem_w1, sem_w2):
    core = lax.axis_index("c")
    sid = lax.axis_index("s")
    wid = sid * NUM_CORES + core
    start = wid * MAIN_CHUNKS + jnp.minimum(wid, BIG_WORKERS)
    sbase = pl.multiple_of(start * CHUNK, CHUNK)

    @pl.when(sid == 0)
    def _():
        pltpu.sync_copy(s01_hbm, s01_sp)
        pltpu.sync_copy(w2_hbm, w2_sp)

    pltpu.sync_copy(c01_hbm.at[pl.ds(sbase, SLOTS_PER_WORKER)], idx01_v)
    pltpu.sync_copy(c2_hbm.at[pl.ds(sbase, SLOTS_PER_WORKER)], idx2_v)
    plsc.subcore_barrier()

    bufs = (buf0, buf1, buf2)
    wsems = (sem_w0, sem_w1, sem_w2)

    def chunk_base(c):
        ob = jnp.minimum((start + c) * CHUNK, LAST_BASE)
        return pl.multiple_of(ob, 32)

    def gather01(c, s):
        off = pl.multiple_of(chunk_base(c) - sbase, 32)
        return pltpu.async_copy(
            s01_sp.at[idx01_v.at[pl.ds(off, CHUNK)]], bufs[s], sem_g)

    def wait_g01(s):
        pltpu.make_async_copy(
            out_hbm.at[pl.ds(0, CHUNK), :], bufs[s], sem_g).wait()

    def gather2_add(c, s):
        off = pl.multiple_of(chunk_base(c) - sbase, 32)
        return pltpu.async_copy(
            w2_sp.at[idx2_v.at[pl.ds(off, CHUNK)]], bufs[s], sem_a, add=True)

    def write(c, s):
        return pltpu.async_copy(
            bufs[s], out_hbm.at[pl.ds(chunk_base(c), CHUNK), :], wsems[s])

    def wait_write(s):
        pltpu.make_async_copy(
            bufs[s], out_hbm.at[pl.ds(0, CHUNK), :], wsems[s]).wait()

    def step(c, s, wait_prev_write):
        wait_g01(s)
        h_a = gather2_add(c, s)
        nxt = (s + 1) % 3
        if wait_prev_write:
            wait_write(nxt)
        gather01(c + 1, nxt)
        h_a.wait()
        write(c, s)

    gather01(0, 0)
    step(0, 0, False)
    step(1, 1, False)
    step(2, 2, True)

    def body(k, carry):
        c = 3 * k
        step(c, 0, True)
        step(c + 1, 1, True)
        step(c + 2, 2, True)
        return carry

    lax.fori_loop(1, 8, body, 0)

    wait_g01(0)

    @pl.when(wid < BIG_WORKERS)
    def _():
        h_a = gather2_add(24, 0)
        h_a.wait()
        write(24, 0)
        wait_write(0)

    wait_write(1)
    wait_write(2)


@jax.jit
def _encode(c01, c2, w2, s01):
    mesh = plsc.VectorSubcoreMesh(core_axis_name="c", subcore_axis_name="s")
    return pl.kernel(
        _sc_body,
        out_type=jax.ShapeDtypeStruct((N_TOTAL, HIDDEN), jnp.float32),
        mesh=mesh,
        scratch_types=[
            pltpu.VMEM((SLOTS_PER_WORKER,), jnp.int32),
            pltpu.VMEM((SLOTS_PER_WORKER,), jnp.int32),
            pltpu.VMEM((CHUNK, HIDDEN), jnp.float32),
            pltpu.VMEM((CHUNK, HIDDEN), jnp.float32),
            pltpu.VMEM((CHUNK, HIDDEN), jnp.float32),
            pltpu.VMEM_SHARED((T * T, HIDDEN), jnp.float32),
            pltpu.VMEM_SHARED((T, HIDDEN), jnp.float32),
            pltpu.SemaphoreType.DMA,
            pltpu.SemaphoreType.DMA,
            pltpu.SemaphoreType.DMA,
            pltpu.SemaphoreType.DMA,
            pltpu.SemaphoreType.DMA,
        ],
    )(c01, c2, w2, s01)


def kernel(x, W0, W1, W2):
    if x.ndim == 1:
        x = x[:, None]
    n = x.shape[0]
    xt = jnp.pad(x.T.astype(jnp.int32), ((0, 0), (0, NP - n)))
    s01, c01, c2 = _combine(W0, W1, W2, xt.reshape(3, NP // HIDDEN, HIDDEN))
    return _encode(c01.reshape(NP), c2.reshape(NP), W2, s01)
